# baseline (speedup 1.0000x reference)
import functools

import jax
import jax.numpy as jnp
from jax import lax
from jax.experimental import pallas as pl
from jax.experimental.pallas import tpu as pltpu
from jax.experimental.pallas import tpu_sc as plsc

_NB = 16384
_OUT_ROWS = _NB * 3
_NW = 32
_RPW = _OUT_ROWS // _NW
_CH = 128
_NCH = _RPW // _CH


def _sc_body(x_hbm, idx_hbm, o_hbm, idx_v, buf0, buf1, sem0, sem1, isem):
    wid = lax.axis_index("s") * 2 + lax.axis_index("c")
    base = wid * _RPW
    pltpu.async_copy(idx_hbm.at[wid], idx_v, isem).wait()

    bufs = (buf0, buf1)
    sems = (sem0, sem1)
    copies = [None, None]
    copies[0] = pltpu.async_copy(x_hbm.at[idx_v.at[0]], buf0, sem0)
    for g in range(_NCH):
        if g + 1 < _NCH:
            copies[(g + 1) % 2] = pltpu.async_copy(
                x_hbm.at[idx_v.at[g + 1]], bufs[(g + 1) % 2], sems[(g + 1) % 2])
        copies[g % 2].wait()
        pltpu.sync_copy(bufs[g % 2], o_hbm.at[pl.ds(base + g * _CH, _CH)])


def kernel(input):
    table = input.reshape(_NB * 32, 128)
    o = jnp.arange(_OUT_ROWS, dtype=jnp.int32)
    idx = (o + 29 * (o // 3) + 15).reshape(_NW, _NCH, _CH)
    k = functools.partial(
        pl.kernel,
        out_type=jax.ShapeDtypeStruct((_OUT_ROWS, 128), jnp.float32),
        mesh=plsc.VectorSubcoreMesh(core_axis_name="c", subcore_axis_name="s"),
        scratch_types=[
            pltpu.VMEM((_NCH, _CH), jnp.int32),
            pltpu.VMEM((_CH, 128), jnp.float32),
            pltpu.VMEM((_CH, 128), jnp.float32),
            pltpu.SemaphoreType.DMA,
            pltpu.SemaphoreType.DMA,
            pltpu.SemaphoreType.DMA,
        ],
    )(_sc_body)
    return k(table, idx).reshape(_NB, 3, 128)


# speedup vs baseline: 1.4974x; 1.4974x over previous
import functools

import jax
import jax.numpy as jnp
from jax import lax
from jax.experimental import pallas as pl
from jax.experimental.pallas import tpu as pltpu
from jax.experimental.pallas import tpu_sc as plsc

_NB = 16384
_NW = 32
_BPW = _NB // _NW
_CHB = 64
_NCH = _BPW // _CHB


def _sc_body(x_hbm, idx_hbm, o_hbm, idx_v, buf0, buf1, sem0, sem1, isem):
    wid = lax.axis_index("s") * 2 + lax.axis_index("c")
    base_b = wid * _BPW
    pltpu.async_copy(idx_hbm.at[wid], idx_v, isem).wait()

    bufs = (buf0, buf1)
    sems = (sem0, sem1)

    def start(g):
        return [
            pltpu.async_copy(
                x_hbm.at[idx_v.at[g, j]], bufs[g % 2].at[:, j, :], sems[g % 2])
            for j in range(3)
        ]

    copies = [None, None]
    copies[0] = start(0)
    for g in range(_NCH):
        if g + 1 < _NCH:
            copies[(g + 1) % 2] = start(g + 1)
        for c in copies[g % 2]:
            c.wait()
        pltpu.sync_copy(
            bufs[g % 2], o_hbm.at[pl.ds(base_b + g * _CHB, _CHB), :, :])


def kernel(input):
    table = input.reshape(_NB * 32, 128)
    b = jnp.arange(_NB, dtype=jnp.int32)
    idx = (32 * b[:, None] + 15 + jnp.arange(3, dtype=jnp.int32)[None, :])
    idx = idx.reshape(_NW, _NCH, _CHB, 3).transpose(0, 1, 3, 2)
    k = functools.partial(
        pl.kernel,
        out_type=jax.ShapeDtypeStruct((_NB, 3, 128), jnp.float32),
        mesh=plsc.VectorSubcoreMesh(core_axis_name="c", subcore_axis_name="s"),
        scratch_types=[
            pltpu.VMEM((_NCH, 3, _CHB), jnp.int32),
            pltpu.VMEM((_CHB, 3, 128), jnp.float32),
            pltpu.VMEM((_CHB, 3, 128), jnp.float32),
            pltpu.SemaphoreType.DMA,
            pltpu.SemaphoreType.DMA,
            pltpu.SemaphoreType.DMA,
        ],
    )(_sc_body)
    return k(table, idx)


# speedup vs baseline: 1.5239x; 1.0177x over previous
import functools

import jax
import jax.numpy as jnp
import numpy as np
from jax import lax
from jax.experimental import pallas as pl
from jax.experimental.pallas import tpu as pltpu
from jax.experimental.pallas import tpu_sc as plsc

_NB = 16384
_NW = 32
_BPW = _NB // _NW
_CHB = 64
_NCH = _BPW // _CHB
_NBUF = 3

_B_OF = (np.arange(_NW)[:, None, None, None] * _BPW
         + np.arange(_NCH)[None, :, None, None] * _CHB
         + np.arange(_CHB)[None, None, None, :])
_IDX = (32 * _B_OF + 15 + np.arange(3)[None, None, :, None]).astype(np.int32)


def _sc_body(x_hbm, idx_hbm, o_hbm, idx_v, *scratch):
    bufs, sems = scratch[:_NBUF], scratch[_NBUF:2 * _NBUF]
    isem = scratch[2 * _NBUF]
    wid = lax.axis_index("s") * 2 + lax.axis_index("c")
    base_b = wid * _BPW
    pltpu.async_copy(idx_hbm.at[wid], idx_v, isem).wait()

    def start(g):
        return [
            pltpu.async_copy(
                x_hbm.at[idx_v.at[g, j]], bufs[g % _NBUF].at[:, j, :],
                sems[g % _NBUF])
            for j in range(3)
        ]

    copies = [None] * _NBUF
    for g in range(_NBUF - 1):
        copies[g] = start(g)
    for g in range(_NCH):
        if g + _NBUF - 1 < _NCH:
            copies[(g + _NBUF - 1) % _NBUF] = start(g + _NBUF - 1)
        for c in copies[g % _NBUF]:
            c.wait()
        pltpu.sync_copy(
            bufs[g % _NBUF], o_hbm.at[pl.ds(base_b + g * _CHB, _CHB), :, :])


def kernel(input):
    table = input.reshape(_NB * 32, 128)
    k = functools.partial(
        pl.kernel,
        out_type=jax.ShapeDtypeStruct((_NB, 3, 128), jnp.float32),
        mesh=plsc.VectorSubcoreMesh(core_axis_name="c", subcore_axis_name="s"),
        scratch_types=(
            [pltpu.VMEM((_NCH, 3, _CHB), jnp.int32)]
            + [pltpu.VMEM((_CHB, 3, 128), jnp.float32)] * _NBUF
            + [pltpu.SemaphoreType.DMA] * _NBUF
            + [pltpu.SemaphoreType.DMA]
        ),
    )(_sc_body)
    return k(table, jnp.asarray(_IDX))


# speedup vs baseline: 1.5279x; 1.0026x over previous
import functools

import jax
import jax.numpy as jnp
import numpy as np
from jax import lax
from jax.experimental import pallas as pl
from jax.experimental.pallas import tpu as pltpu
from jax.experimental.pallas import tpu_sc as plsc

_NB = 16384
_NW = 32
_BPW = _NB // _NW
_CHB = 64
_NCH = _BPW // _CHB
_NBUF = 3

_B_OF = (np.arange(_NW)[:, None, None, None] * _BPW
         + np.arange(_NCH)[None, :, None, None] * _CHB
         + np.arange(_CHB)[None, None, None, :])
_IDX = (32 * _B_OF + 15 + np.arange(3)[None, None, :, None]).astype(np.int32)


def _sc_body(x_hbm, idx_hbm, o_hbm, idx_v, *scratch):
    bufs, sems = scratch[:_NBUF], scratch[_NBUF:2 * _NBUF]
    isem = scratch[2 * _NBUF]
    wid = lax.axis_index("c") * 16 + lax.axis_index("s")
    base_b = wid * _BPW
    pltpu.async_copy(idx_hbm.at[wid], idx_v, isem).wait()

    def start(g):
        return [
            pltpu.async_copy(
                x_hbm.at[idx_v.at[g, j]], bufs[g % _NBUF].at[:, j, :],
                sems[g % _NBUF])
            for j in range(3)
        ]

    copies = [None] * _NBUF
    for g in range(_NBUF - 1):
        copies[g] = start(g)
    for g in range(_NCH):
        if g + _NBUF - 1 < _NCH:
            copies[(g + _NBUF - 1) % _NBUF] = start(g + _NBUF - 1)
        for c in copies[g % _NBUF]:
            c.wait()
        pltpu.sync_copy(
            bufs[g % _NBUF], o_hbm.at[pl.ds(base_b + g * _CHB, _CHB), :, :])


def kernel(input):
    table = input.reshape(_NB * 32, 128)
    k = functools.partial(
        pl.kernel,
        out_type=jax.ShapeDtypeStruct((_NB, 3, 128), jnp.float32),
        mesh=plsc.VectorSubcoreMesh(core_axis_name="c", subcore_axis_name="s"),
        scratch_types=(
            [pltpu.VMEM((_NCH, 3, _CHB), jnp.int32)]
            + [pltpu.VMEM((_CHB, 3, 128), jnp.float32)] * _NBUF
            + [pltpu.SemaphoreType.DMA] * _NBUF
            + [pltpu.SemaphoreType.DMA]
        ),
    )(_sc_body)
    return k(table, jnp.asarray(_IDX))
